# baseline (speedup 1.0000x reference)
import functools
import math

import jax
import jax.numpy as jnp
from jax.experimental import pallas as pl
from jax.experimental.pallas import tpu as pltpu

_DIM = 64
_BS = 1024
_B = 1024
_M = 2048
_TOP_K = 16
_TEMPERATURE = 5.0
_ROWS = 32
_GROUP = 4


def _controller_kernel(pmt_ref, state_ref, wq_ref, bq_ref, wk_ref, bk_ref,
                       best_ref, logits_ref, idx_ref, w_ref):
    f32 = jnp.float32
    bf16 = jnp.bfloat16
    q = jax.lax.dot_general(
        state_ref[...].astype(bf16), wq_ref[...].astype(bf16),
        (((1,), (1,)), ((), ())),
        preferred_element_type=f32) + bq_ref[...]
    q_b = q.astype(bf16)
    wk_b = wk_ref[...].astype(bf16)
    bk = bk_ref[...]
    row_g = jax.lax.broadcasted_iota(jnp.int32, (_GROUP, 1), 0)
    inv = 1.0 / math.sqrt(_DIM)

    s_groups = []
    for g0 in range(0, _ROWS, _GROUP):
        kparts = []
        for r in range(g0, g0 + _GROUP):
            pmr_b = pmt_ref[r].astype(bf16)
            kT = jax.lax.dot_general(
                wk_b, pmr_b, (((1,), (0,)), ((), ())),
                preferred_element_type=f32) + bk
            kparts.append(kT.astype(bf16))
        kcat = jnp.concatenate(kparts, axis=1)
        sr_all = jax.lax.dot_general(
            q_b[g0:g0 + _GROUP, :], kcat, (((1,), (0,)), ((), ())),
            preferred_element_type=f32) * inv
        sg = None
        for r in range(_GROUP):
            strip = sr_all[:, r * _M:(r + 1) * _M]
            sg = strip if sg is None else jnp.where(row_g == r, strip, sg)
        s_groups.append(sg)
    s = jnp.concatenate(s_groups, axis=0)
    logits_ref[...] = s

    iota_m = jax.lax.broadcasted_iota(jnp.int32, (_ROWS, _M), 1)
    neg_inf = jnp.float32(-jnp.inf)

    def fold(cur):
        pairs = [(cur[:, t * 128:(t + 1) * 128],
                  iota_m[:, t * 128:(t + 1) * 128]) for t in range(_M // 128)]
        while len(pairs) > 1:
            nxt = []
            for i in range(0, len(pairs), 2):
                (av, am), (bv, bm) = pairs[i], pairs[i + 1]
                c = bv > av
                nxt.append((jnp.where(c, bv, av), jnp.where(c, bm, am)))
            pairs = nxt
        return pairs[0]

    vals = []
    idxs = []
    cur = s
    for _ in range(_TOP_K):
        vf, mi = fold(cur)
        mx = jnp.max(vf, axis=1, keepdims=True)
        idx = jnp.min(jnp.where(vf == mx, mi, _M), axis=1, keepdims=True)
        cur = jnp.where(iota_m == idx, neg_inf, cur)
        vals.append(mx)
        idxs.append(idx)
    tv = jnp.concatenate(vals, axis=1)
    ti = jnp.concatenate(idxs, axis=1)
    e = jnp.exp((tv - tv[:, :1]) / _TEMPERATURE)
    w_ref[...] = e / jnp.sum(e, axis=1, keepdims=True)
    idx_ref[...] = ti
    best_ref[...] = ti[:, :1] * _BS


@functools.partial(jax.jit, static_argnames=())
def kernel(pm, state, Wq, bq, Wk, bk):
    grid = _B // _ROWS
    out_shapes = (
        jax.ShapeDtypeStruct((_B, 1), jnp.int32),
        jax.ShapeDtypeStruct((_B, _M), jnp.float32),
        jax.ShapeDtypeStruct((_B, _TOP_K), jnp.int32),
        jax.ShapeDtypeStruct((_B, _TOP_K), jnp.float32),
    )
    pmt = pm.transpose(0, 2, 1)
    best2d, logits, topk_idx, topk_w = pl.pallas_call(
        _controller_kernel,
        grid=(grid,),
        in_specs=[
            pl.BlockSpec((_ROWS, _DIM, _M), lambda g: (g, 0, 0)),
            pl.BlockSpec((_ROWS, _DIM), lambda g: (g, 0)),
            pl.BlockSpec((_DIM, _DIM), lambda g: (0, 0)),
            pl.BlockSpec((1, _DIM), lambda g: (0, 0)),
            pl.BlockSpec((_DIM, _DIM), lambda g: (0, 0)),
            pl.BlockSpec((_DIM, 1), lambda g: (0, 0)),
        ],
        out_specs=(
            pl.BlockSpec((_ROWS, 1), lambda g: (g, 0)),
            pl.BlockSpec((_ROWS, _M), lambda g: (g, 0)),
            pl.BlockSpec((_ROWS, _TOP_K), lambda g: (g, 0)),
            pl.BlockSpec((_ROWS, _TOP_K), lambda g: (g, 0)),
        ),
        out_shape=out_shapes,
        compiler_params=pltpu.CompilerParams(
            dimension_semantics=("arbitrary",),
        ),
    )(pmt, state, Wq, bq.reshape(1, _DIM), Wk, bk.reshape(_DIM, 1))
    return best2d.reshape(_B), logits, topk_idx, topk_w


# speedup vs baseline: 1.0096x; 1.0096x over previous
import functools
import math

import jax
import jax.numpy as jnp
from jax.experimental import pallas as pl
from jax.experimental.pallas import tpu as pltpu

_DIM = 64
_BS = 1024
_B = 1024
_M = 2048
_TOP_K = 16
_TEMPERATURE = 5.0
_ROWS = 32
_GROUP = 8


def _controller_kernel(pmt_ref, state_ref, wq_ref, bq_ref, wk_ref, bk_ref,
                       best_ref, logits_ref, idx_ref, w_ref):
    f32 = jnp.float32
    bf16 = jnp.bfloat16
    q = jax.lax.dot_general(
        state_ref[...].astype(bf16), wq_ref[...].astype(bf16),
        (((1,), (1,)), ((), ())),
        preferred_element_type=f32) + bq_ref[...]
    q_b = q.astype(bf16)
    wk_b = wk_ref[...].astype(bf16)
    bk = bk_ref[...]
    row_g = jax.lax.broadcasted_iota(jnp.int32, (_GROUP, 1), 0)
    inv = 1.0 / math.sqrt(_DIM)

    s_groups = []
    for g0 in range(0, _ROWS, _GROUP):
        kparts = []
        for r in range(g0, g0 + _GROUP):
            pmr_b = pmt_ref[r].astype(bf16)
            kT = jax.lax.dot_general(
                wk_b, pmr_b, (((1,), (0,)), ((), ())),
                preferred_element_type=f32) + bk
            kparts.append(kT.astype(bf16))
        kcat = jnp.concatenate(kparts, axis=1)
        sr_all = jax.lax.dot_general(
            q_b[g0:g0 + _GROUP, :], kcat, (((1,), (0,)), ((), ())),
            preferred_element_type=f32) * inv
        sg = None
        for r in range(_GROUP):
            strip = sr_all[:, r * _M:(r + 1) * _M]
            sg = strip if sg is None else jnp.where(row_g == r, strip, sg)
        s_groups.append(sg)
    s = jnp.concatenate(s_groups, axis=0)
    logits_ref[...] = s

    iota_m = jax.lax.broadcasted_iota(jnp.int32, (_ROWS, _M), 1)
    neg_inf = jnp.float32(-jnp.inf)

    def fold(cur):
        pairs = [(cur[:, t * 128:(t + 1) * 128],
                  iota_m[:, t * 128:(t + 1) * 128]) for t in range(_M // 128)]
        while len(pairs) > 1:
            nxt = []
            for i in range(0, len(pairs), 2):
                (av, am), (bv, bm) = pairs[i], pairs[i + 1]
                c = bv > av
                nxt.append((jnp.where(c, bv, av), jnp.where(c, bm, am)))
            pairs = nxt
        return pairs[0]

    vals = []
    idxs = []
    cur = s
    for _ in range(_TOP_K):
        vf, mi = fold(cur)
        mx = jnp.max(vf, axis=1, keepdims=True)
        idx = jnp.min(jnp.where(vf == mx, mi, _M), axis=1, keepdims=True)
        cur = jnp.where(iota_m == idx, neg_inf, cur)
        vals.append(mx)
        idxs.append(idx)
    tv = jnp.concatenate(vals, axis=1)
    ti = jnp.concatenate(idxs, axis=1)
    e = jnp.exp((tv - tv[:, :1]) / _TEMPERATURE)
    w_ref[...] = e / jnp.sum(e, axis=1, keepdims=True)
    idx_ref[...] = ti
    best_ref[...] = ti[:, :1] * _BS


@functools.partial(jax.jit, static_argnames=())
def kernel(pm, state, Wq, bq, Wk, bk):
    grid = _B // _ROWS
    out_shapes = (
        jax.ShapeDtypeStruct((_B, 1), jnp.int32),
        jax.ShapeDtypeStruct((_B, _M), jnp.float32),
        jax.ShapeDtypeStruct((_B, _TOP_K), jnp.int32),
        jax.ShapeDtypeStruct((_B, _TOP_K), jnp.float32),
    )
    pmt = pm.transpose(0, 2, 1)
    best2d, logits, topk_idx, topk_w = pl.pallas_call(
        _controller_kernel,
        grid=(grid,),
        in_specs=[
            pl.BlockSpec((_ROWS, _DIM, _M), lambda g: (g, 0, 0)),
            pl.BlockSpec((_ROWS, _DIM), lambda g: (g, 0)),
            pl.BlockSpec((_DIM, _DIM), lambda g: (0, 0)),
            pl.BlockSpec((1, _DIM), lambda g: (0, 0)),
            pl.BlockSpec((_DIM, _DIM), lambda g: (0, 0)),
            pl.BlockSpec((_DIM, 1), lambda g: (0, 0)),
        ],
        out_specs=(
            pl.BlockSpec((_ROWS, 1), lambda g: (g, 0)),
            pl.BlockSpec((_ROWS, _M), lambda g: (g, 0)),
            pl.BlockSpec((_ROWS, _TOP_K), lambda g: (g, 0)),
            pl.BlockSpec((_ROWS, _TOP_K), lambda g: (g, 0)),
        ),
        out_shape=out_shapes,
        compiler_params=pltpu.CompilerParams(
            dimension_semantics=("arbitrary",),
        ),
    )(pmt, state, Wq, bq.reshape(1, _DIM), Wk, bk.reshape(_DIM, 1))
    return best2d.reshape(_B), logits, topk_idx, topk_w
